# baseline (speedup 1.0000x reference)
import jax
import jax.numpy as jnp
from jax import lax
from jax.experimental import pallas as pl
from jax.experimental.pallas import tpu as pltpu
from jax.experimental.pallas import tpu_sc as plsc

B, N, F, K = 4, 4096, 128, 512
NC, NS = 2, 16
NW = NC * NS
ROWS_PER_W = (B * K) // NW
CH = 8
LANES = 16


def _topk_tc_kernel(x_ref, p_ref, idxloc_ref, idxflat_ref, gate_ref):
    b = pl.program_id(0)
    xb = x_ref[0]
    pv = p_ref[...]
    nrm = jnp.sqrt(jnp.sum(pv * pv))
    s_col = jnp.dot(xb, pv, preferred_element_type=jnp.float32) / (nrm + 1e-7)
    s_row = lax.dot_general(pv, xb, (((0,), (1,)), ((), ())),
                            preferred_element_type=jnp.float32) / (nrm + 1e-7)
    cnts = []
    TI = 512
    for t in range(N // TI):
        si = lax.slice(s_row, (0, t * TI), (1, (t + 1) * TI))
        gt = s_col > si
        eq = s_col == si
        jj = lax.broadcasted_iota(jnp.int32, (N, TI), 0)
        ii = lax.broadcasted_iota(jnp.int32, (N, TI), 1) + t * TI
        sel = jnp.where(gt | (eq & (jj < ii)), 1.0, 0.0)
        cnts.append(jnp.sum(sel, axis=0, keepdims=True))
    rank_row = jnp.concatenate(cnts, axis=1)
    rr = lax.broadcasted_iota(jnp.int32, (K, N), 0)
    Sb = rank_row.astype(jnp.int32) == rr
    ii_n = lax.broadcasted_iota(jnp.int32, (K, N), 1)
    idx_col = jnp.sum(jnp.where(Sb, ii_n, 0), axis=1, keepdims=True)
    vals_col = jnp.sum(jnp.where(Sb, s_row, 0.0), axis=1, keepdims=True)
    idxloc_ref[...] = idx_col.reshape(1, K, 1)
    idxflat_ref[...] = (idx_col + b * N).reshape(1, K, 1)
    gate_ref[...] = jnp.tanh(vals_col).reshape(1, K, 1)


def _topk_tc(x, p):
    return pl.pallas_call(
        _topk_tc_kernel,
        grid=(B,),
        in_specs=[
            pl.BlockSpec((1, N, F), lambda b: (b, 0, 0)),
            pl.BlockSpec((F, 1), lambda b: (0, 0)),
        ],
        out_specs=[
            pl.BlockSpec((1, K, 1), lambda b: (b, 0, 0)),
            pl.BlockSpec((1, K, 1), lambda b: (b, 0, 0)),
            pl.BlockSpec((1, K, 1), lambda b: (b, 0, 0)),
        ],
        out_shape=[
            jax.ShapeDtypeStruct((B, K, 1), jnp.int32),
            jax.ShapeDtypeStruct((B, K, 1), jnp.int32),
            jax.ShapeDtypeStruct((B, K, 1), jnp.float32),
        ],
    )(x, p)


def _gather_sc_kernel(A2, X2, idxflat, idxloc, gate, outA, outX,
                      rowids_v, colidx_v, gate_v, rows_v, xrows_v,
                      outbufA_v, outbufX_v, semA, semX):
    wid = lax.axis_index("s") * NC + lax.axis_index("c")
    base_rows = wid * ROWS_PER_W
    col_off = pl.multiple_of((base_rows // K) * K, K)
    pltpu.sync_copy(idxloc.at[pl.ds(col_off, K)], colidx_v)
    pltpu.sync_copy(idxflat.at[pl.ds(base_rows, ROWS_PER_W)], rowids_v)
    pltpu.sync_copy(gate.at[pl.ds(base_rows, ROWS_PER_W)], gate_v)
    lanes = lax.iota(jnp.int32, LANES)

    def chunk_body(c, carry):
        row0 = pl.multiple_of(base_rows + c * CH, CH)
        idref = rowids_v.at[pl.ds(c * CH, CH)]
        cpA = pltpu.async_copy(A2.at[idref], rows_v, semA)
        cpX = pltpu.async_copy(X2.at[idref], xrows_v, semX)
        cpX.wait()
        for r in range(CH):
            gv = plsc.load_gather(gate_v, [jnp.zeros((LANES,), jnp.int32)
                                           + (c * CH + r)])
            rsel = jnp.full((LANES,), r, jnp.int32)
            for f in range(F // LANES):
                xc = plsc.load_gather(xrows_v, [rsel, lanes + f * LANES])
                outbufX_v[pl.ds(r * F + f * LANES, LANES)] = xc * gv
        cpA.wait()
        for r in range(CH):
            rsel = jnp.full((LANES,), r, jnp.int32)
            for j in range(K // LANES):
                idxv = colidx_v[pl.ds(j * LANES, LANES)]
                vals = plsc.load_gather(rows_v, [rsel, idxv])
                outbufA_v[pl.ds(r * K + j * LANES, LANES)] = vals
        pltpu.sync_copy(outbufA_v, outA.at[pl.ds(row0 * K, CH * K)])
        pltpu.sync_copy(outbufX_v, outX.at[pl.ds(row0 * F, CH * F)])
        return carry

    lax.fori_loop(0, ROWS_PER_W // CH, chunk_body, None)


def _gather_sc(A2, X2, idxflat, idxloc, gate):
    mesh = plsc.VectorSubcoreMesh(core_axis_name="c", subcore_axis_name="s")
    return pl.kernel(
        _gather_sc_kernel,
        out_type=[jax.ShapeDtypeStruct((B * K * K,), jnp.float32),
                  jax.ShapeDtypeStruct((B * K * F,), jnp.float32)],
        mesh=mesh,
        compiler_params=pltpu.CompilerParams(use_tc_tiling_on_sc=False,
                                             needs_layout_passes=False),
        scratch_types=[
            pltpu.VMEM((ROWS_PER_W,), jnp.int32),
            pltpu.VMEM((K,), jnp.int32),
            pltpu.VMEM((ROWS_PER_W,), jnp.float32),
            pltpu.VMEM((CH, N), jnp.float32),
            pltpu.VMEM((CH, F), jnp.float32),
            pltpu.VMEM((CH * K,), jnp.float32),
            pltpu.VMEM((CH * F,), jnp.float32),
            pltpu.SemaphoreType.DMA,
            pltpu.SemaphoreType.DMA,
        ],
    )(A2, X2, idxflat, idxloc, gate)


@jax.jit
def kernel(A, x, p):
    idxloc, idxflat, gate = _topk_tc(x, p)
    A2 = A.reshape(B * N, N)
    X2 = x.reshape(B * N, F)
    outA, outX = _gather_sc(A2, X2, idxflat.reshape(B * K),
                            idxloc.reshape(B * K), gate.reshape(B * K))
    return outA.reshape(B, K, K), outX.reshape(B, K, F)


# speedup vs baseline: 2.2766x; 2.2766x over previous
import jax
import jax.numpy as jnp
from jax import lax
from jax.experimental import pallas as pl
from jax.experimental.pallas import tpu as pltpu
from jax.experimental.pallas import tpu_sc as plsc

B, N, F, K = 4, 4096, 128, 512
NC, NS = 2, 16
NW = NC * NS
ROWS_PER_W = (B * K) // NW
CH = 8
LANES = 16


def _topk_tc_kernel(x_ref, p_ref, idxloc_ref, idxflat_ref, gate_ref):
    b = pl.program_id(0)
    xb = x_ref[0]
    pv = p_ref[...]
    nrm = jnp.sqrt(jnp.sum(pv * pv))
    s_col = jnp.dot(xb, pv, preferred_element_type=jnp.float32) / (nrm + 1e-7)
    s_row = lax.dot_general(pv, xb, (((0,), (1,)), ((), ())),
                            preferred_element_type=jnp.float32) / (nrm + 1e-7)
    cnts = []
    TI = 512
    for t in range(N // TI):
        si = lax.slice(s_row, (0, t * TI), (1, (t + 1) * TI))
        gt = s_col > si
        eq = s_col == si
        jj = lax.broadcasted_iota(jnp.int32, (N, TI), 0)
        ii = lax.broadcasted_iota(jnp.int32, (N, TI), 1) + t * TI
        sel = jnp.where(gt | (eq & (jj < ii)), 1.0, 0.0)
        cnts.append(jnp.sum(sel, axis=0, keepdims=True))
    rank_row = jnp.concatenate(cnts, axis=1)
    rr = lax.broadcasted_iota(jnp.int32, (K, N), 0)
    Sb = rank_row.astype(jnp.int32) == rr
    ii_n = lax.broadcasted_iota(jnp.int32, (K, N), 1)
    idx_col = jnp.sum(jnp.where(Sb, ii_n, 0), axis=1, keepdims=True)
    vals_col = jnp.sum(jnp.where(Sb, s_row, 0.0), axis=1, keepdims=True)
    idxloc_ref[...] = idx_col.reshape(1, K, 1)
    idxflat_ref[...] = (idx_col + b * N).reshape(1, K, 1)
    gate_ref[...] = jnp.tanh(vals_col).reshape(1, K, 1)


def _topk_tc(x, p):
    return pl.pallas_call(
        _topk_tc_kernel,
        grid=(B,),
        in_specs=[
            pl.BlockSpec((1, N, F), lambda b: (b, 0, 0)),
            pl.BlockSpec((F, 1), lambda b: (0, 0)),
        ],
        out_specs=[
            pl.BlockSpec((1, K, 1), lambda b: (b, 0, 0)),
            pl.BlockSpec((1, K, 1), lambda b: (b, 0, 0)),
            pl.BlockSpec((1, K, 1), lambda b: (b, 0, 0)),
        ],
        out_shape=[
            jax.ShapeDtypeStruct((B, K, 1), jnp.int32),
            jax.ShapeDtypeStruct((B, K, 1), jnp.int32),
            jax.ShapeDtypeStruct((B, K, 1), jnp.float32),
        ],
    )(x, p)


def _gather_sc_kernel(A2, X2, idxflat, idxloc, gate, outA, outX,
                      rowids_v, colidx_v, gate_v, rows_v, xrows_v,
                      outbufA_v, outbufX_v, semA, semX):
    wid = lax.axis_index("s") * NC + lax.axis_index("c")
    base_rows = wid * ROWS_PER_W
    col_off = pl.multiple_of((base_rows // K) * K, K)
    pltpu.sync_copy(idxloc.at[pl.ds(col_off, K)], colidx_v)
    pltpu.sync_copy(idxflat.at[pl.ds(base_rows, ROWS_PER_W)], rowids_v)
    pltpu.sync_copy(gate.at[pl.ds(base_rows, ROWS_PER_W)], gate_v)
    lanes = lax.iota(jnp.int32, LANES)

    def chunk_body(c, carry):
        row0 = pl.multiple_of(base_rows + c * CH, CH)
        idref = rowids_v.at[pl.ds(c * CH, CH)]
        cpA = pltpu.async_copy(A2.at[idref], rows_v, semA)
        cpX = pltpu.async_copy(X2.at[idref], xrows_v, semX)
        cpX.wait()
        for r in range(CH):
            gv = plsc.load_gather(gate_v, [jnp.zeros((LANES,), jnp.int32)
                                           + (c * CH + r)])
            rsel = jnp.full((LANES,), r, jnp.int32)
            for f in range(F // LANES):
                xc = plsc.load_gather(xrows_v, [rsel, lanes + f * LANES])
                outbufX_v[pl.ds(r * F + f * LANES, LANES)] = xc * gv
        cpA.wait()
        for r in range(CH):
            rsel = jnp.full((LANES,), r, jnp.int32)
            for j in range(K // LANES):
                idxv = colidx_v[pl.ds(j * LANES, LANES)]
                vals = plsc.load_gather(rows_v, [rsel, idxv])
                outbufA_v[pl.ds(r * K + j * LANES, LANES)] = vals
        pltpu.sync_copy(outbufA_v, outA.at[pl.ds(row0 * K, CH * K)])
        pltpu.sync_copy(outbufX_v, outX.at[pl.ds(row0 * F, CH * F)])
        return carry

    lax.fori_loop(0, ROWS_PER_W // CH, chunk_body, None)


def _gather_sc(A2, X2, idxflat, idxloc, gate):
    mesh = plsc.VectorSubcoreMesh(core_axis_name="c", subcore_axis_name="s")
    return pl.kernel(
        _gather_sc_kernel,
        out_type=[jax.ShapeDtypeStruct((B * K * K,), jnp.float32),
                  jax.ShapeDtypeStruct((B * K * F,), jnp.float32)],
        mesh=mesh,
        compiler_params=pltpu.CompilerParams(use_tc_tiling_on_sc=True,
                                             needs_layout_passes=False),
        scratch_types=[
            pltpu.VMEM((ROWS_PER_W,), jnp.int32),
            pltpu.VMEM((K,), jnp.int32),
            pltpu.VMEM((ROWS_PER_W,), jnp.float32),
            pltpu.VMEM((CH, N), jnp.float32),
            pltpu.VMEM((CH, F), jnp.float32),
            pltpu.VMEM((CH * K,), jnp.float32),
            pltpu.VMEM((CH * F,), jnp.float32),
            pltpu.SemaphoreType.DMA,
            pltpu.SemaphoreType.DMA,
        ],
    )(A2, X2, idxflat, idxloc, gate)


@jax.jit
def kernel(A, x, p):
    idxloc, idxflat, gate = _topk_tc(x, p)
    A2 = A.reshape(B * N, N)
    X2 = x.reshape(B * N, F)
    outA, outX = _gather_sc(A2, X2, idxflat.reshape(B * K),
                            idxloc.reshape(B * K), gate.reshape(B * K))
    return outA.reshape(B, K, K), outX.reshape(B, K, F)
